# baseline (speedup 1.0000x reference)
import functools

import jax
import jax.numpy as jnp
from jax import lax
from jax.experimental import pallas as pl
from jax.experimental.pallas import tpu as pltpu
from jax.experimental.pallas import tpu_sc as plsc

_MIN_DIST_THRESH = 0.05
_FAR = 1e18
_CW = 256


def _dist_hit_body(params_ref, cloth_ref, saug_ref, dmin_ref, hit_ref, *,
                   nsp, nc, cw):
    nt = nsp // cw
    cb = cloth_ref[0, :, :].astype(jnp.bfloat16)
    c0 = jnp.broadcast_to(cb[:, 0:1], (nc, cw))
    c1 = jnp.broadcast_to(cb[:, 1:2], (nc, cw))
    c2 = jnp.broadcast_to(cb[:, 2:3], (nc, cw))
    ci0 = params_ref[0:1, 2:3]
    ci1 = params_ref[0:1, 3:4]
    mint2 = jnp.bfloat16(_MIN_DIST_THRESH * _MIN_DIST_THRESH)
    sent = jnp.bfloat16(1e8)

    def chunk_step(t, carry):
        run_min, run_hit = carry
        off = t * cw
        s0 = saug_ref[0, 0:1, pl.ds(off, cw)]
        s1 = saug_ref[0, 1:2, pl.ds(off, cw)]
        s2 = saug_ref[0, 2:3, pl.ds(off, cw)]
        idxf = saug_ref[0, 3:4, pl.ds(off, cw)]
        sv = saug_ref[0, 4:5, pl.ds(off, cw)]
        far = jnp.float32(_FAR)
        s0 = jnp.where(sv > 0.0, s0, far).astype(jnp.bfloat16)
        s1 = jnp.where(sv > 0.0, s1, far).astype(jnp.bfloat16)
        s2 = jnp.where(sv > 0.0, s2, far).astype(jnp.bfloat16)
        hitc = jnp.logical_or(idxf == ci0, idxf == ci1).astype(jnp.bfloat16)
        d0 = s0 - c0
        d1 = s1 - c1
        d2 = s2 - c2
        dsq = d0 * d0 + d1 * d1 + d2 * d2
        dsq = jnp.where(dsq < mint2, sent, dsq)
        upd = dsq < run_min
        return (jnp.where(upd, dsq, run_min), jnp.where(upd, hitc, run_hit))

    init = (jnp.full((nc, cw), jnp.finfo(jnp.bfloat16).max, jnp.bfloat16),
            jnp.zeros((nc, cw), jnp.bfloat16))
    run_min, run_hit = jax.lax.fori_loop(0, nt, chunk_step, init)

    dmin = jnp.min(run_min, axis=1, keepdims=True)
    eq = run_min == dmin
    hit = jnp.min(jnp.where(eq, run_hit, jnp.bfloat16(1.0)),
                  axis=1, keepdims=True).astype(jnp.float32)
    dmin_ref[...] = dmin.astype(jnp.float32)[None]
    hit_ref[...] = hit[None]


def _sc_loss_reduce(dmin_flat, hit_flat, sdf_flat, params, b, nc):
    nw = 32
    rows = (b * nc) // nw
    groups = rows // 16
    mesh = plsc.VectorSubcoreMesh(core_axis_name="c", subcore_axis_name="s")

    @functools.partial(
        pl.kernel, mesh=mesh,
        out_type=(jax.ShapeDtypeStruct((nw, 16), jnp.float32),
                  jax.ShapeDtypeStruct((nw, 16), jnp.float32)),
        scratch_types=[
            pltpu.VMEM((rows,), jnp.float32),
            pltpu.VMEM((rows,), jnp.float32),
            pltpu.VMEM((rows,), jnp.float32),
            pltpu.VMEM((4, 16), jnp.float32),
            pltpu.VMEM((16,), jnp.float32),
            pltpu.VMEM((16,), jnp.float32),
        ],
    )
    def sc_kernel(dmin_hbm, hit_hbm, sdf_hbm, params_hbm,
                  loss_out, any_out,
                  dmin_v, hit_v, sdf_v, params_v, acc_v, any_v):
        wid = lax.axis_index("s") * 2 + lax.axis_index("c")
        base = wid * rows
        pltpu.sync_copy(dmin_hbm.at[pl.ds(base, rows)], dmin_v)
        pltpu.sync_copy(hit_hbm.at[pl.ds(base, rows)], hit_v)
        pltpu.sync_copy(sdf_hbm.at[pl.ds(base, rows)], sdf_v)
        pltpu.sync_copy(params_hbm, params_v)
        stv = params_v[0]
        dt2v = params_v[1]
        acc = jnp.zeros((16,), jnp.float32)
        anyh = jnp.zeros((16,), jnp.float32)
        zero = jnp.zeros((16,), jnp.float32)
        for g in range(groups):
            dmin2 = dmin_v[pl.ds(g * 16, 16)]
            hitf = hit_v[pl.ds(g * 16, 16)]
            s16 = sdf_v[pl.ds(g * 16, 16)]
            within = dmin2 < dt2v
            term = jnp.abs(s16) * hitf + jnp.abs(s16 - stv) * (1.0 - hitf)
            acc = acc + jnp.where(within, term, zero)
            anyh = jnp.maximum(anyh, hitf)
        acc_v[...] = acc
        any_v[...] = anyh
        pltpu.sync_copy(acc_v, loss_out.at[wid])
        pltpu.sync_copy(any_v, any_out.at[wid])

    return sc_kernel(dmin_flat, hit_flat, sdf_flat, params)


def kernel(sdf, cloth_meshes_unposed, smpl_cloth_idx, smpl_cloth_valid,
           cloth_idx, sdf_thresh, dist_thresh, v_template):
    b, nc = sdf.shape
    ns = v_template.shape[1]
    nsp = ((ns + _CW - 1) // _CW) * _CW

    idxf = smpl_cloth_idx.astype(jnp.float32)[..., None]
    validf = (smpl_cloth_valid > 0).astype(jnp.float32)[..., None]
    saug = jnp.concatenate(
        [v_template.astype(jnp.float32), idxf, validf], axis=-1)
    saug = jnp.pad(saug, ((0, 0), (0, nsp - ns), (0, 0)))
    saug_t = jnp.transpose(saug, (0, 2, 1))
    params = jnp.zeros((1, 128), jnp.float32)
    params = params.at[0, 0].set(jnp.asarray(sdf_thresh, jnp.float32))
    params = params.at[0, 1].set(jnp.asarray(dist_thresh, jnp.float32))
    params = params.at[0, 2].set(cloth_idx[0].astype(jnp.float32))
    params = params.at[0, 3].set(cloth_idx[1].astype(jnp.float32))

    body = functools.partial(_dist_hit_body, nsp=nsp, nc=nc, cw=_CW)
    dmin, hit = pl.pallas_call(
        body,
        grid=(b,),
        in_specs=[
            pl.BlockSpec((1, 128), lambda i: (0, 0)),
            pl.BlockSpec((1, nc, 3), lambda i: (i, 0, 0)),
            pl.BlockSpec((1, 5, nsp), lambda i: (i, 0, 0)),
        ],
        out_specs=(pl.BlockSpec((1, nc, 1), lambda i: (i, 0, 0)),
                   pl.BlockSpec((1, nc, 1), lambda i: (i, 0, 0))),
        out_shape=(jax.ShapeDtypeStruct((b, nc, 1), jnp.float32),
                   jax.ShapeDtypeStruct((b, nc, 1), jnp.float32)),
    )(params, cloth_meshes_unposed, saug_t)

    scparams = jnp.stack([
        jnp.full((16,), jnp.asarray(sdf_thresh, jnp.float32)),
        jnp.full((16,), jnp.asarray(dist_thresh, jnp.float32) ** 2),
        jnp.zeros((16,), jnp.float32),
        jnp.zeros((16,), jnp.float32),
    ])
    loss_p, any_p = _sc_loss_reduce(dmin.reshape(b * nc), hit.reshape(b * nc),
                                    sdf.reshape(b * nc), scparams, b, nc)
    loss_p = loss_p.reshape(b, -1)
    any_p = any_p.reshape(b, -1)
    exist = (jnp.max(any_p, axis=1) > 0.0).astype(jnp.float32)
    return jnp.sum(loss_p, axis=1) / jnp.float32(nc) * exist


# speedup vs baseline: 1.3712x; 1.3712x over previous
import functools

import jax
import jax.numpy as jnp
from jax import lax
from jax.experimental import pallas as pl
from jax.experimental.pallas import tpu as pltpu
from jax.experimental.pallas import tpu_sc as plsc

_MIN_DIST_THRESH = 0.05
_FAR = 1e18
_CT = 512


def _dist_hit_body(params_ref, cloth_ref, saug_ref, dmin_ref, hit_ref, *,
                   nsp, nc, ct):
    nt = nsp // ct
    c0 = cloth_ref[0, 0:1, :].astype(jnp.bfloat16)
    c1 = cloth_ref[0, 1:2, :].astype(jnp.bfloat16)
    c2 = cloth_ref[0, 2:3, :].astype(jnp.bfloat16)
    ci0 = params_ref[0:1, 2:3]
    ci1 = params_ref[0:1, 3:4]
    mint2 = jnp.bfloat16(_MIN_DIST_THRESH * _MIN_DIST_THRESH)
    sent = jnp.bfloat16(1e8)
    oneb = jnp.bfloat16(1.0)

    def tile_step(t, carry):
        run_min, run_hit = carry
        off = t * ct
        s3 = saug_ref[0, pl.ds(off, ct), 0:3]
        idxf = saug_ref[0, pl.ds(off, ct), 3:4]
        sv = saug_ref[0, pl.ds(off, ct), 4:5]
        s3 = jnp.where(sv > 0.0, s3, jnp.float32(_FAR)).astype(jnp.bfloat16)
        hitc = jnp.logical_or(idxf == ci0, idxf == ci1).astype(jnp.bfloat16)
        d0 = s3[:, 0:1] - c0
        d1 = s3[:, 1:2] - c1
        d2 = s3[:, 2:3] - c2
        dsq = d0 * d0 + d1 * d1 + d2 * d2
        dsq = jnp.where(dsq < mint2, sent, dsq)
        tmin = jnp.min(dsq, axis=0, keepdims=True)
        eq = dsq == tmin
        hitt = jnp.min(jnp.where(eq, hitc, oneb), axis=0, keepdims=True)
        upd = tmin < run_min
        return (jnp.minimum(run_min, tmin), jnp.where(upd, hitt, run_hit))

    init = (jnp.full((1, nc), jnp.finfo(jnp.bfloat16).max, jnp.bfloat16),
            jnp.zeros((1, nc), jnp.bfloat16))
    run_min, run_hit = jax.lax.fori_loop(0, nt, tile_step, init)
    dmin_ref[...] = run_min.astype(jnp.float32)[None]
    hit_ref[...] = run_hit.astype(jnp.float32)[None]


def _sc_loss_reduce(dmin_flat, hit_flat, sdf_flat, params, b, nc):
    nw = 32
    rows = (b * nc) // nw
    groups = rows // 16
    mesh = plsc.VectorSubcoreMesh(core_axis_name="c", subcore_axis_name="s")

    @functools.partial(
        pl.kernel, mesh=mesh,
        out_type=(jax.ShapeDtypeStruct((nw, 16), jnp.float32),
                  jax.ShapeDtypeStruct((nw, 16), jnp.float32)),
        scratch_types=[
            pltpu.VMEM((rows,), jnp.float32),
            pltpu.VMEM((rows,), jnp.float32),
            pltpu.VMEM((rows,), jnp.float32),
            pltpu.VMEM((4, 16), jnp.float32),
            pltpu.VMEM((16,), jnp.float32),
            pltpu.VMEM((16,), jnp.float32),
        ],
    )
    def sc_kernel(dmin_hbm, hit_hbm, sdf_hbm, params_hbm,
                  loss_out, any_out,
                  dmin_v, hit_v, sdf_v, params_v, acc_v, any_v):
        wid = lax.axis_index("s") * 2 + lax.axis_index("c")
        base = wid * rows
        pltpu.sync_copy(dmin_hbm.at[pl.ds(base, rows)], dmin_v)
        pltpu.sync_copy(hit_hbm.at[pl.ds(base, rows)], hit_v)
        pltpu.sync_copy(sdf_hbm.at[pl.ds(base, rows)], sdf_v)
        pltpu.sync_copy(params_hbm, params_v)
        stv = params_v[0]
        dt2v = params_v[1]
        acc = jnp.zeros((16,), jnp.float32)
        anyh = jnp.zeros((16,), jnp.float32)
        zero = jnp.zeros((16,), jnp.float32)
        for g in range(groups):
            dmin2 = dmin_v[pl.ds(g * 16, 16)]
            hitf = hit_v[pl.ds(g * 16, 16)]
            s16 = sdf_v[pl.ds(g * 16, 16)]
            within = dmin2 < dt2v
            term = jnp.abs(s16) * hitf + jnp.abs(s16 - stv) * (1.0 - hitf)
            acc = acc + jnp.where(within, term, zero)
            anyh = jnp.maximum(anyh, hitf)
        acc_v[...] = acc
        any_v[...] = anyh
        pltpu.sync_copy(acc_v, loss_out.at[wid])
        pltpu.sync_copy(any_v, any_out.at[wid])

    return sc_kernel(dmin_flat, hit_flat, sdf_flat, params)


def kernel(sdf, cloth_meshes_unposed, smpl_cloth_idx, smpl_cloth_valid,
           cloth_idx, sdf_thresh, dist_thresh, v_template):
    b, nc = sdf.shape
    ns = v_template.shape[1]
    nsp = ((ns + _CT - 1) // _CT) * _CT

    idxf = smpl_cloth_idx.astype(jnp.float32)[..., None]
    validf = (smpl_cloth_valid > 0).astype(jnp.float32)[..., None]
    saug = jnp.concatenate(
        [v_template.astype(jnp.float32), idxf, validf], axis=-1)
    saug = jnp.pad(saug, ((0, 0), (0, nsp - ns), (0, 0)))
    cloth_t = jnp.transpose(cloth_meshes_unposed, (0, 2, 1))
    params = jnp.zeros((1, 128), jnp.float32)
    params = params.at[0, 0].set(jnp.asarray(sdf_thresh, jnp.float32))
    params = params.at[0, 1].set(jnp.asarray(dist_thresh, jnp.float32) ** 2)
    params = params.at[0, 2].set(cloth_idx[0].astype(jnp.float32))
    params = params.at[0, 3].set(cloth_idx[1].astype(jnp.float32))

    body = functools.partial(_dist_hit_body, nsp=nsp, nc=nc, ct=_CT)
    dmin, hit = pl.pallas_call(
        body,
        grid=(b,),
        in_specs=[
            pl.BlockSpec((1, 128), lambda i: (0, 0)),
            pl.BlockSpec((1, 3, nc), lambda i: (i, 0, 0)),
            pl.BlockSpec((1, nsp, 5), lambda i: (i, 0, 0)),
        ],
        out_specs=(pl.BlockSpec((1, 1, nc), lambda i: (i, 0, 0)),
                   pl.BlockSpec((1, 1, nc), lambda i: (i, 0, 0))),
        out_shape=(jax.ShapeDtypeStruct((b, 1, nc), jnp.float32),
                   jax.ShapeDtypeStruct((b, 1, nc), jnp.float32)),
    )(params, cloth_t, saug)

    scparams = jnp.stack([
        jnp.full((16,), jnp.asarray(sdf_thresh, jnp.float32)),
        jnp.full((16,), jnp.asarray(dist_thresh, jnp.float32) ** 2),
        jnp.zeros((16,), jnp.float32),
        jnp.zeros((16,), jnp.float32),
    ])
    loss_p, any_p = _sc_loss_reduce(dmin.reshape(b * nc), hit.reshape(b * nc),
                                    sdf.reshape(b * nc), scparams, b, nc)
    loss_p = loss_p.reshape(b, -1)
    any_p = any_p.reshape(b, -1)
    exist = (jnp.max(any_p, axis=1) > 0.0).astype(jnp.float32)
    return jnp.sum(loss_p, axis=1) / jnp.float32(nc) * exist
